# baseline (speedup 1.0000x reference)
import jax
import jax.numpy as jnp
from jax import lax
from jax.experimental import pallas as pl
from jax.experimental.pallas import tpu as pltpu
from jax.experimental.pallas import tpu_sc as plsc

_B = 1024
_V = 100000
_TBL = 128
_NC, _NS = 2, 16
_NW = _NC * _NS
_RPW = _B // _NW
_CB = 20000
_CPR = _V // _CB
_NIT = _RPW * _CPR
_UNROLL = 10


def _fma_chunk(xb, nb, ob, c1, c2):
    def body(j, carry):
        base = j * (_UNROLL * 16)
        for u in range(_UNROLL):
            sl = pl.ds(base + u * 16, 16)
            ob[sl] = c1 * xb[sl] + c2 * nb[sl]
        return carry
    lax.fori_loop(0, _CB // 16 // _UNROLL, body, 0)


def _body(x_hbm, n_hbm, t_hbm, sac_hbm, somac_hbm, o_hbm,
          xb0, xb1, nb0, nb1, ob0, ob1,
          tv, sacv, somacv, c1v, c2v,
          sx0, sx1, sn0, sn1, so0, so1):
    wid = lax.axis_index("s") * _NC + lax.axis_index("c")
    row_base = wid * _RPW

    pltpu.sync_copy(t_hbm.at[pl.ds(row_base, _RPW)], tv)
    pltpu.sync_copy(sac_hbm, sacv)
    pltpu.sync_copy(somac_hbm, somacv)
    for g in range(_RPW // 16):
        idx = tv[pl.ds(g * 16, 16)]
        c1v[pl.ds(g * 16, 16)] = plsc.load_gather(sacv, [idx])
        c2v[pl.ds(g * 16, 16)] = plsc.load_gather(somacv, [idx])

    def elem_off(i):
        return (row_base + i // _CPR) * _V + (i % _CPR) * _CB

    def start_in(i, xb, nb, sx, sn):
        off = elem_off(i)
        pltpu.make_async_copy(x_hbm.at[pl.ds(off, _CB)], xb, sx).start()
        pltpu.make_async_copy(n_hbm.at[pl.ds(off, _CB)], nb, sn).start()

    def wait_in(xb, nb, sx, sn):
        pltpu.make_async_copy(x_hbm.at[pl.ds(0, _CB)], xb, sx).wait()
        pltpu.make_async_copy(n_hbm.at[pl.ds(0, _CB)], nb, sn).wait()

    def wait_out(ob, so):
        pltpu.make_async_copy(ob, o_hbm.at[pl.ds(0, _CB)], so).wait()

    start_in(0, xb0, nb0, sx0, sn0)
    start_in(1, xb1, nb1, sx1, sn1)

    bufs = ((xb0, nb0, ob0, sx0, sn0, so0),
            (xb1, nb1, ob1, sx1, sn1, so1))

    def outer(g, carry):
        for b in range(2):
            xb, nb, ob, sx, sn, so = bufs[b]
            i = g * 2 + b
            wait_in(xb, nb, sx, sn)

            @pl.when(g > 0)
            def _():
                wait_out(ob, so)

            r_local = i // _CPR
            idx16 = jnp.full((16,), r_local, jnp.int32)
            c1 = plsc.load_gather(c1v, [idx16])
            c2 = plsc.load_gather(c2v, [idx16])
            _fma_chunk(xb, nb, ob, c1, c2)

            pltpu.make_async_copy(
                ob, o_hbm.at[pl.ds(elem_off(i), _CB)], so).start()

            @pl.when(i + 2 < _NIT)
            def _():
                start_in(i + 2, xb, nb, sx, sn)
        return carry

    lax.fori_loop(0, _NIT // 2, outer, 0)
    wait_out(ob0, so0)
    wait_out(ob1, so1)


def kernel(x_start, noise, sqrt_alphas_cumprod, sqrt_one_minus_alphas_cumprod, t):
    B, V = x_start.shape
    xf = x_start.reshape(B * V)
    nf = noise.reshape(B * V)
    nsteps = sqrt_alphas_cumprod.shape[0]
    sac = jnp.zeros((_TBL,), jnp.float32).at[:nsteps].set(sqrt_alphas_cumprod)
    somac = jnp.zeros((_TBL,), jnp.float32).at[:nsteps].set(
        sqrt_one_minus_alphas_cumprod)

    mesh = plsc.VectorSubcoreMesh(
        core_axis_name="c", subcore_axis_name="s",
        num_cores=_NC, num_subcores=_NS)
    out = pl.kernel(
        _body,
        mesh=mesh,
        out_type=jax.ShapeDtypeStruct((B * V,), jnp.float32),
        compiler_params=pltpu.CompilerParams(needs_layout_passes=False),
        scratch_types=[
            pltpu.VMEM((_CB,), jnp.float32),
            pltpu.VMEM((_CB,), jnp.float32),
            pltpu.VMEM((_CB,), jnp.float32),
            pltpu.VMEM((_CB,), jnp.float32),
            pltpu.VMEM((_CB,), jnp.float32),
            pltpu.VMEM((_CB,), jnp.float32),
            pltpu.VMEM((_RPW,), jnp.int32),
            pltpu.VMEM((_TBL,), jnp.float32),
            pltpu.VMEM((_TBL,), jnp.float32),
            pltpu.VMEM((_RPW,), jnp.float32),
            pltpu.VMEM((_RPW,), jnp.float32),
            pltpu.SemaphoreType.DMA,
            pltpu.SemaphoreType.DMA,
            pltpu.SemaphoreType.DMA,
            pltpu.SemaphoreType.DMA,
            pltpu.SemaphoreType.DMA,
            pltpu.SemaphoreType.DMA,
        ],
    )(xf, nf, t, sac, somac)
    return out.reshape(B, V)


# speedup vs baseline: 10.1105x; 10.1105x over previous
import jax
import jax.numpy as jnp
from jax.experimental import pallas as pl
from jax.experimental.pallas import tpu as pltpu

_STEPS = 100
_BR = 1024


def _fma_body(t_ref, sac_ref, somac_ref, x_ref, n_ref, o_ref, c1_ref, c2_ref):
    @pl.when(pl.program_id(0) == 0)
    def _gather_coeffs():
        trow = t_ref[...]
        b = trow.shape[1]
        steps = jax.lax.broadcasted_iota(jnp.int32, (_STEPS, b), 0)
        m = trow == steps
        c1_ref[...] = jnp.sum(
            jnp.where(m, sac_ref[...], 0.0), axis=0, keepdims=True
        )
        c2_ref[...] = jnp.sum(
            jnp.where(m, somac_ref[...], 0.0), axis=0, keepdims=True
        )

    o_ref[...] = c1_ref[...] * x_ref[...] + c2_ref[...] * n_ref[...]


def kernel(x_start, noise, sqrt_alphas_cumprod, sqrt_one_minus_alphas_cumprod, t):
    B, V = x_start.shape
    xt = x_start.T
    nt = noise.T
    t2 = t.reshape(1, B)
    sac2 = sqrt_alphas_cumprod.reshape(_STEPS, 1)
    somac2 = sqrt_one_minus_alphas_cumprod.reshape(_STEPS, 1)

    out_t = pl.pallas_call(
        _fma_body,
        grid=(pl.cdiv(V, _BR),),
        in_specs=[
            pl.BlockSpec((1, B), lambda j: (0, 0)),
            pl.BlockSpec((_STEPS, 1), lambda j: (0, 0)),
            pl.BlockSpec((_STEPS, 1), lambda j: (0, 0)),
            pl.BlockSpec((_BR, B), lambda j: (j, 0)),
            pl.BlockSpec((_BR, B), lambda j: (j, 0)),
        ],
        out_specs=pl.BlockSpec((_BR, B), lambda j: (j, 0)),
        out_shape=jax.ShapeDtypeStruct((V, B), x_start.dtype),
        scratch_shapes=[
            pltpu.VMEM((1, B), jnp.float32),
            pltpu.VMEM((1, B), jnp.float32),
        ],
        compiler_params=pltpu.CompilerParams(
            dimension_semantics=("arbitrary",),
        ),
    )(t2, sac2, somac2, xt, nt)
    return out_t.T
